# baseline (speedup 1.0000x reference)
import functools

import jax
import jax.numpy as jnp
from jax import lax
from jax.experimental import pallas as pl
from jax.experimental.pallas import tpu as pltpu
from jax.experimental.pallas import tpu_sc as plsc

N = 10000
E = 320000
D = 128

NC = 2
NS = 16
NW = NC * NS
CHUNK = 64
NCHUNK = 160
EPWP = NCHUNK * CHUNK
EPAD = NW * EPWP
NP = 10240
RPW = NP // NS
NBUF = 5
NROUND = NCHUNK // NBUF


def _seg_sum_body(src_hbm, dst_hbm, h_hbm, out_hbm,
                  sidx_v, didx_v, rows_v, isem, gsem, ssem, acc_sh):
    c = lax.axis_index("c")
    s = lax.axis_index("s")
    wid = s * NC + c
    ebase = wid * EPWP

    def zrow(i, carry):
        for j in range(D // 16):
            rows_v[0, i, pl.ds(j * 16, 16)] = jnp.zeros((16,), jnp.float32)
        return carry
    lax.fori_loop(0, CHUNK, zrow, 0)
    for t in range(RPW // CHUNK):
        pltpu.sync_copy(rows_v.at[0], acc_sh.at[pl.ds(s * RPW + t * CHUNK, CHUNK)])
    plsc.subcore_barrier()

    def issue_idx(k, p, b):
        base = pl.multiple_of(ebase + k * CHUNK, 8)
        pltpu.async_copy(src_hbm.at[pl.ds(base, CHUNK)], sidx_v.at[p, b], isem.at[p, b])
        pltpu.async_copy(dst_hbm.at[pl.ds(base, CHUNK)], didx_v.at[p, b], isem.at[p, b])

    def wait_idx(p, b):
        pltpu.make_async_copy(src_hbm.at[pl.ds(0, CHUNK)], sidx_v.at[p, b], isem.at[p, b]).wait()
        pltpu.make_async_copy(dst_hbm.at[pl.ds(0, CHUNK)], didx_v.at[p, b], isem.at[p, b]).wait()

    for j in range(2 * NBUF):
        issue_idx(j, j // NBUF, j % NBUF)
    for b in range(NBUF):
        wait_idx(0, b)
        pltpu.async_copy(h_hbm.at[sidx_v.at[0, b]], rows_v.at[b], gsem.at[b])

    def round_body(r2, carry):
        for half in range(2):
            p = half
            for b in range(NBUF):
                k = (2 * r2 + half) * NBUF + b
                if b > 0:
                    pp, pb = p, b - 1
                else:
                    pp, pb = 1 - p, NBUF - 1
                pltpu.make_async_copy(h_hbm.at[sidx_v.at[p, b]], rows_v.at[b],
                                      gsem.at[b]).wait()
                pltpu.async_copy(rows_v.at[b], acc_sh.at[didx_v.at[p, b]],
                                 ssem.at[b], add=True)

                @pl.when(k >= 1)
                def _():
                    pltpu.make_async_copy(rows_v.at[pb],
                                          acc_sh.at[didx_v.at[pp, pb]],
                                          ssem.at[pb]).wait()

                    @pl.when(k - 1 < NCHUNK - 2 * NBUF)
                    def _():
                        issue_idx(k - 1 + 2 * NBUF, pp, pb)

                    @pl.when(k - 1 < NCHUNK - NBUF)
                    def _():
                        wait_idx(1 - pp, pb)
                        pltpu.async_copy(h_hbm.at[sidx_v.at[1 - pp, pb]],
                                         rows_v.at[pb], gsem.at[pb])
        return carry
    lax.fori_loop(0, NROUND // 2, round_body, 0)

    pltpu.make_async_copy(rows_v.at[NBUF - 1],
                          acc_sh.at[didx_v.at[(NROUND - 1) % 2, NBUF - 1]],
                          ssem.at[NBUF - 1]).wait()

    plsc.subcore_barrier()
    row0 = pl.multiple_of(c * NP + s * RPW, RPW)
    pltpu.sync_copy(acc_sh.at[pl.ds(s * RPW, RPW)], out_hbm.at[pl.ds(row0, RPW)])


@jax.jit
def _segment_sum_sc(src_p, dst_p, h):
    mesh = plsc.VectorSubcoreMesh(core_axis_name="c", subcore_axis_name="s")
    f = pl.kernel(
        _seg_sum_body,
        out_type=jax.ShapeDtypeStruct((NC * NP, D), jnp.float32),
        mesh=mesh,
        scratch_types=[
            pltpu.VMEM((2, NBUF, CHUNK), jnp.int32),
            pltpu.VMEM((2, NBUF, CHUNK), jnp.int32),
            pltpu.VMEM((NBUF, CHUNK, D), jnp.float32),
            pltpu.SemaphoreType.DMA((2, NBUF)),
            pltpu.SemaphoreType.DMA((NBUF,)),
            pltpu.SemaphoreType.DMA((NBUF,)),
            pltpu.VMEM_SHARED((NP, D), jnp.float32),
        ],
    )
    return f(src_p, dst_p, h)


R = 2000
GRID = N // R


def _root_body(h_ref, wroot_ref, b_ref, res_ref, o_ref):
    y = jnp.dot(h_ref[...], wroot_ref[...], preferred_element_type=jnp.float32,
                precision=lax.Precision.HIGHEST)
    y += b_ref[...]
    if res_ref is not None:
        y += res_ref[...]
    o_ref[...] = y


@jax.jit
def _root_tc(h, w_root, b, res=None):
    body = _root_body if res is not None else (
        lambda h_ref, wroot_ref, b_ref, o_ref:
        _root_body(h_ref, wroot_ref, b_ref, None, o_ref))
    args = (h, w_root, b.reshape(1, D))
    in_specs = [
        pl.BlockSpec((R, D), lambda i: (i, 0)),
        pl.BlockSpec((D, D), lambda i: (0, 0)),
        pl.BlockSpec((1, D), lambda i: (0, 0)),
    ]
    if res is not None:
        args = args + (res,)
        in_specs.append(pl.BlockSpec((R, D), lambda i: (i, 0)))
    return pl.pallas_call(
        body,
        grid=(GRID,),
        in_specs=in_specs,
        out_specs=pl.BlockSpec((R, D), lambda i: (i, 0)),
        out_shape=jax.ShapeDtypeStruct((N, D), jnp.float32),
    )(*args)


def _combine_body(p_ref, root_ref, wrel_ref, o_ref):
    agg = p_ref[0] + p_ref[1]
    y = jnp.dot(agg, wrel_ref[...], preferred_element_type=jnp.float32,
                precision=lax.Precision.HIGHEST)
    y += root_ref[...]
    o_ref[...] = jnp.where(y >= 0, y, 0.01 * y)


@jax.jit
def _combine_tc(p, root, w_rel):
    return pl.pallas_call(
        _combine_body,
        grid=(GRID,),
        in_specs=[
            pl.BlockSpec((NC, R, D), lambda i: (0, i, 0)),
            pl.BlockSpec((R, D), lambda i: (i, 0)),
            pl.BlockSpec((D, D), lambda i: (0, 0)),
        ],
        out_specs=pl.BlockSpec((R, D), lambda i: (i, 0)),
        out_shape=jax.ShapeDtypeStruct((N, D), jnp.float32),
    )(p.reshape(NC, NP, D), root, w_rel)


def kernel(x, edge_index, W1_rel, b1, W1_root, W2_rel, b2, W2_root):
    npad = EPAD - E
    pad_iota = jnp.arange(npad, dtype=jnp.int32)
    src = jnp.concatenate([edge_index[0], pad_iota % N])
    dst = jnp.concatenate([edge_index[1], N + pad_iota % (NP - N)])
    p1 = _segment_sum_sc(src, dst, x)
    root1 = _root_tc(x, W1_root, b1)
    h1 = _combine_tc(p1, root1, W1_rel)
    p2 = _segment_sum_sc(src, dst, h1)
    root2 = _root_tc(h1, W2_root, b2, res=x)
    out = _combine_tc(p2, root2, W2_rel)
    return out


# speedup vs baseline: 1.0399x; 1.0399x over previous
import functools

import jax
import jax.numpy as jnp
from jax import lax
from jax.experimental import pallas as pl
from jax.experimental.pallas import tpu as pltpu
from jax.experimental.pallas import tpu_sc as plsc

N = 10000
E = 320000
D = 128

NC = 2
NS = 16
NW = NC * NS
CHUNK = 40
NCHUNK = 250
EPWP = NCHUNK * CHUNK
EPAD = NW * EPWP
NP = 10240
RPW = NP // NS
NBUF = 5
NROUND = NCHUNK // NBUF


def _seg_sum_body(src_hbm, dst_hbm, h_hbm, out_hbm,
                  sidx_v, didx_v, rows_v, isem, gsem, acc_sh):
    c = lax.axis_index("c")
    s = lax.axis_index("s")
    wid = s * NC + c
    ebase = wid * EPWP

    def zrow(i, carry):
        for j in range(D // 16):
            rows_v[0, i, pl.ds(j * 16, 16)] = jnp.zeros((16,), jnp.float32)
        return carry
    lax.fori_loop(0, CHUNK, zrow, 0)
    for t in range(RPW // CHUNK):
        pltpu.sync_copy(rows_v.at[0], acc_sh.at[pl.ds(s * RPW + t * CHUNK, CHUNK)])
    plsc.subcore_barrier()

    def issue_idx(k, p, b):
        base = pl.multiple_of(ebase + k * CHUNK, 8)
        pltpu.async_copy(src_hbm.at[pl.ds(base, CHUNK)], sidx_v.at[p, b], isem.at[p, b])
        pltpu.async_copy(dst_hbm.at[pl.ds(base, CHUNK)], didx_v.at[p, b], isem.at[p, b])

    def wait_idx(p, b):
        pltpu.make_async_copy(src_hbm.at[pl.ds(0, CHUNK)], sidx_v.at[p, b], isem.at[p, b]).wait()
        pltpu.make_async_copy(dst_hbm.at[pl.ds(0, CHUNK)], didx_v.at[p, b], isem.at[p, b]).wait()

    for j in range(2 * NBUF):
        issue_idx(j, j // NBUF, j % NBUF)
    for b in range(NBUF):
        wait_idx(0, b)
        pltpu.async_copy(h_hbm.at[sidx_v.at[0, b]], rows_v.at[b], gsem.at[b])

    def round_body(r2, carry):
        for half in range(2):
            p = half
            q = 1 - half
            for b in range(NBUF):
                k = (2 * r2 + half) * NBUF + b
                pltpu.make_async_copy(h_hbm.at[sidx_v.at[p, b]], rows_v.at[b],
                                      gsem.at[b]).wait()
                pltpu.sync_copy(rows_v.at[b], acc_sh.at[didx_v.at[p, b]], add=True)

                @pl.when(k < NCHUNK - 2 * NBUF)
                def _():
                    issue_idx(k + 2 * NBUF, p, b)

                @pl.when(k < NCHUNK - NBUF)
                def _():
                    wait_idx(q, b)
                    pltpu.async_copy(h_hbm.at[sidx_v.at[q, b]], rows_v.at[b],
                                     gsem.at[b])
        return carry
    lax.fori_loop(0, NROUND // 2, round_body, 0)

    plsc.subcore_barrier()
    row0 = pl.multiple_of(c * NP + s * RPW, RPW)
    pltpu.sync_copy(acc_sh.at[pl.ds(s * RPW, RPW)], out_hbm.at[pl.ds(row0, RPW)])


@jax.jit
def _segment_sum_sc(src_p, dst_p, h):
    mesh = plsc.VectorSubcoreMesh(core_axis_name="c", subcore_axis_name="s")
    f = pl.kernel(
        _seg_sum_body,
        out_type=jax.ShapeDtypeStruct((NC * NP, D), jnp.float32),
        mesh=mesh,
        scratch_types=[
            pltpu.VMEM((2, NBUF, CHUNK), jnp.int32),
            pltpu.VMEM((2, NBUF, CHUNK), jnp.int32),
            pltpu.VMEM((NBUF, CHUNK, D), jnp.float32),
            pltpu.SemaphoreType.DMA((2, NBUF)),
            pltpu.SemaphoreType.DMA((NBUF,)),
            pltpu.VMEM_SHARED((NP, D), jnp.float32),
        ],
    )
    return f(src_p, dst_p, h)


R = 2000
GRID = N // R


def _root_body(h_ref, wroot_ref, b_ref, res_ref, o_ref):
    y = jnp.dot(h_ref[...], wroot_ref[...], preferred_element_type=jnp.float32,
                precision=lax.Precision.HIGHEST)
    y += b_ref[...]
    if res_ref is not None:
        y += res_ref[...]
    o_ref[...] = y


@jax.jit
def _root_tc(h, w_root, b, res=None):
    body = _root_body if res is not None else (
        lambda h_ref, wroot_ref, b_ref, o_ref:
        _root_body(h_ref, wroot_ref, b_ref, None, o_ref))
    args = (h, w_root, b.reshape(1, D))
    in_specs = [
        pl.BlockSpec((R, D), lambda i: (i, 0)),
        pl.BlockSpec((D, D), lambda i: (0, 0)),
        pl.BlockSpec((1, D), lambda i: (0, 0)),
    ]
    if res is not None:
        args = args + (res,)
        in_specs.append(pl.BlockSpec((R, D), lambda i: (i, 0)))
    return pl.pallas_call(
        body,
        grid=(GRID,),
        in_specs=in_specs,
        out_specs=pl.BlockSpec((R, D), lambda i: (i, 0)),
        out_shape=jax.ShapeDtypeStruct((N, D), jnp.float32),
    )(*args)


def _combine_body(p_ref, root_ref, wrel_ref, o_ref):
    agg = p_ref[0] + p_ref[1]
    y = jnp.dot(agg, wrel_ref[...], preferred_element_type=jnp.float32,
                precision=lax.Precision.HIGHEST)
    y += root_ref[...]
    o_ref[...] = jnp.where(y >= 0, y, 0.01 * y)


@jax.jit
def _combine_tc(p, root, w_rel):
    return pl.pallas_call(
        _combine_body,
        grid=(GRID,),
        in_specs=[
            pl.BlockSpec((NC, R, D), lambda i: (0, i, 0)),
            pl.BlockSpec((R, D), lambda i: (i, 0)),
            pl.BlockSpec((D, D), lambda i: (0, 0)),
        ],
        out_specs=pl.BlockSpec((R, D), lambda i: (i, 0)),
        out_shape=jax.ShapeDtypeStruct((N, D), jnp.float32),
    )(p.reshape(NC, NP, D), root, w_rel)


def kernel(x, edge_index, W1_rel, b1, W1_root, W2_rel, b2, W2_root):
    npad = EPAD - E
    pad_iota = jnp.arange(npad, dtype=jnp.int32)
    src = jnp.concatenate([edge_index[0], pad_iota % N])
    dst = jnp.concatenate([edge_index[1], N + pad_iota % (NP - N)])
    p1 = _segment_sum_sc(src, dst, x)
    root1 = _root_tc(x, W1_root, b1)
    h1 = _combine_tc(p1, root1, W1_rel)
    p2 = _segment_sum_sc(src, dst, h1)
    root2 = _root_tc(h1, W2_root, b2, res=x)
    out = _combine_tc(p2, root2, W2_rel)
    return out
